# baseline (speedup 1.0000x reference)
import functools

import jax
import jax.numpy as jnp
from jax import lax
from jax.experimental import pallas as pl
from jax.experimental.pallas import tpu as pltpu
from jax.experimental.pallas import tpu_sc as plsc

N = 10000
E = 320000
D_IN = 128
D_MID = 128
D_MID2 = 64
D_OUT = 128

NC = 2
NS = 16
TILES = NC * NS
E_PER_TILE = E // TILES
RS = 624
TAIL = N - NS * RS
CH = 128
N_CHUNKS = E_PER_TILE // CH
ECH_TAIL = E_PER_TILE - N_CHUNKS * CH
NB = 5
CHA = 64
NCHA = E_PER_TILE // CHA
AGG_TAIL = E_PER_TILE - NCHA * CHA
PK = NCHA * CHA // 2
ZR = 16
NZ = RS // ZR

_mesh = lambda: plsc.VectorSubcoreMesh(core_axis_name="c", subcore_axis_name="s")


def _sc_degree(dst):
    @functools.partial(
        pl.kernel,
        mesh=_mesh(),
        out_type=jax.ShapeDtypeStruct((NC, N), jnp.float32),
        scratch_types=[
            pltpu.VMEM((E_PER_TILE,), jnp.int32),
            pltpu.VMEM((CH,), jnp.int32),
            pltpu.VMEM((CH,), jnp.int32),
            pltpu.VMEM((ECH_TAIL,), jnp.int32),
            pltpu.VMEM((CH,), jnp.float32),
            pltpu.VMEM((1008,), jnp.float32),
            pltpu.VMEM_SHARED((N,), jnp.float32),
            pltpu.SemaphoreType.DMA,
            pltpu.SemaphoreType.DMA,
        ],
    )
    def k(dst_hbm, out_hbm, slab, dca, dcb, dct, ones, zbuf, acc, sem_a, sem_b):
        c = lax.axis_index("c")
        s = lax.axis_index("s")
        wid = c * NS + s

        slab_cp = pltpu.async_copy(dst_hbm.at[wid], slab, sem_a)

        def zfill(j, carry):
            zbuf[pl.ds(j * 16, 16)] = jnp.zeros((16,), jnp.float32)
            return carry

        lax.fori_loop(0, 63, zfill, 0)

        def ofill(j, carry):
            ones[pl.ds(j * 16, 16)] = jnp.ones((16,), jnp.float32)
            return carry

        lax.fori_loop(0, CH // 16, ofill, 0)

        @pl.when(s < 10)
        def _():
            pltpu.sync_copy(zbuf.at[pl.ds(0, 1000)], acc.at[pl.ds(s * 1000, 1000)])

        slab_cp.wait()
        plsc.subcore_barrier()

        def cpidx(i, buf, n):
            def one(j, carry):
                buf[pl.ds(j * 16, 16)] = slab[pl.ds(i * CH + j * 16, 16)]
                return carry

            lax.fori_loop(0, n // 16, one, 0)

        def scat_start(i, buf, sem):
            cpidx(i, buf, CH)
            pltpu.async_copy(ones, acc.at[buf], sem, add=True)

        def scat_wait(buf, sem):
            pltpu.make_async_copy(dst_hbm.at[wid, pl.ds(0, CH)], buf, sem).wait()

        scat_start(0, dca, sem_a)

        def body(j, carry):
            i0 = 2 * j
            scat_start(i0 + 1, dcb, sem_b)
            scat_wait(dca, sem_a)

            @pl.when(i0 + 2 < N_CHUNKS)
            def _():
                scat_start(i0 + 2, dca, sem_a)

            scat_wait(dcb, sem_b)
            return carry

        lax.fori_loop(0, N_CHUNKS // 2, body, 0)
        dct[pl.ds(0, 16)] = slab[pl.ds(N_CHUNKS * CH, 16)]
        pltpu.sync_copy(ones.at[pl.ds(0, ECH_TAIL)], acc.at[dct], add=True)
        plsc.subcore_barrier()

        @pl.when(s == 0)
        def _():
            pltpu.sync_copy(acc, out_hbm.at[c])

    return k(dst)


def _make_sc_agg(d):
    nb = NB

    @functools.partial(
        pl.kernel,
        mesh=_mesh(),
        out_type=jax.ShapeDtypeStruct((NC, N, d), jnp.float32),
        scratch_types=(
            [pltpu.VMEM((PK,), jnp.int32)]
            + [pltpu.VMEM((CHA,), jnp.int32)] * nb
            + [pltpu.VMEM((CHA,), jnp.int32)] * nb
            + [pltpu.VMEM((AGG_TAIL,), jnp.int32)] * 2
            + [pltpu.VMEM((CHA, d), jnp.float32)] * nb
            + [pltpu.VMEM((ZR, d), jnp.float32)]
            + [pltpu.VMEM_SHARED((N, d), jnp.float32)]
            + [pltpu.SemaphoreType.DMA] * (2 * nb + 1)
        ),
    )
    def k(y_hbm, src_hbm, src1_hbm, dst_hbm, out_hbm, *rest):
        srcv = rest[0]
        sbufs = rest[1:1 + nb]
        dbufs = rest[1 + nb:1 + 2 * nb]
        sct, dct = rest[1 + 2 * nb:3 + 2 * nb]
        rows = rest[3 + 2 * nb:3 + 3 * nb]
        zbuf = rest[3 + 3 * nb]
        acc = rest[4 + 3 * nb]
        gsems = rest[5 + 3 * nb:5 + 4 * nb]
        dsems = rest[5 + 4 * nb:5 + 5 * nb]
        zsem = rest[5 + 5 * nb]
        c = lax.axis_index("c")
        s = lax.axis_index("s")
        wid = c * NS + s

        idx_cp_s = pltpu.async_copy(src_hbm.at[wid], srcv, gsems[0])

        def zrow(r, carry):
            def zcol(j, carry2):
                zbuf[r, pl.ds(j * 16, 16)] = jnp.zeros((16,), jnp.float32)
                return carry2

            return lax.fori_loop(0, d // 16, zcol, carry)

        lax.fori_loop(0, ZR, zrow, 0)

        def zstart(t, carry):
            pltpu.async_copy(zbuf, acc.at[pl.ds(s * RS + t * ZR, ZR)], zsem)
            return carry

        lax.fori_loop(0, NZ, zstart, 0)

        @pl.when(s == 0)
        def _():
            pltpu.sync_copy(zbuf.at[pl.ds(0, TAIL)], acc.at[pl.ds(NS * RS, TAIL)])

        def zdrain(t, carry):
            pltpu.make_async_copy(y_hbm.at[pl.ds(0, ZR)], zbuf, zsem).wait()
            return carry

        lax.fori_loop(0, NZ, zdrain, 0)
        idx_cp_s.wait()
        plsc.subcore_barrier()

        def cpidx(i, buf):
            def one(g, carry):
                v = srcv[pl.ds(i * (CHA // 2) + g * 16, 16)]
                buf[pl.ds(g * 16, 16)] = v & 0xFFFF
                buf[pl.ds(CHA // 2 + g * 16, 16)] = v >> 16
                return carry

            lax.fori_loop(0, CHA // 32, one, 0)

        def stage(i, b):
            cpidx(i, sbufs[b])
            pltpu.async_copy(y_hbm.at[sbufs[b]], rows[b], gsems[b])
            pltpu.async_copy(
                dst_hbm.at[pl.ds(wid * E_PER_TILE + i * CHA, CHA)],
                dbufs[b], dsems[b])

        def finish(b):
            pltpu.make_async_copy(y_hbm.at[pl.ds(0, CHA)], rows[b], gsems[b]).wait()
            pltpu.make_async_copy(
                dst_hbm.at[pl.ds(0, CHA)], dbufs[b], dsems[b]).wait()
            pltpu.sync_copy(rows[b], acc.at[dbufs[b]], add=True)

        for b in range(nb - 1):
            stage(b, b)

        def body(j, carry):
            i0 = nb * j
            stage(i0 + nb - 1, nb - 1)
            for b in range(nb):
                finish(b)
                if b < nb - 1:
                    @pl.when(i0 + nb + b < NCHA)
                    def _(b=b, i0=i0):
                        stage(i0 + nb + b, b)
            return carry

        lax.fori_loop(0, NCHA // nb, body, 0)
        for r in range((NCHA // nb) * nb, NCHA):
            finish(r % nb)

        pltpu.sync_copy(
            src1_hbm.at[pl.ds(wid * E_PER_TILE + NCHA * CHA, AGG_TAIL)], sct)
        pltpu.sync_copy(
            dst_hbm.at[pl.ds(wid * E_PER_TILE + NCHA * CHA, AGG_TAIL)], dct)
        pltpu.async_copy(
            y_hbm.at[sct], rows[0].at[pl.ds(0, AGG_TAIL)], gsems[0]).wait()
        pltpu.sync_copy(rows[0].at[pl.ds(0, AGG_TAIL)], acc.at[dct], add=True)
        plsc.subcore_barrier()

        pltpu.sync_copy(
            acc.at[pl.ds(s * RS, RS)],
            out_hbm.at[c, pl.ds(s * RS, RS), :],
        )

        @pl.when(s == 0)
        def _():
            pltpu.sync_copy(
                acc.at[pl.ds(NS * RS, TAIL)],
                out_hbm.at[c, pl.ds(NS * RS, TAIL), :],
            )

    return k


_sc_agg128 = _make_sc_agg(D_MID)


BN = 2000
GRID = N // BN


def _elu(a):
    return jnp.where(a > 0.0, a, jnp.exp(jnp.minimum(a, 0.0)) - 1.0)


def _rows(i):
    return (i, 0)


def _fixed(i):
    return (0, 0)


def _rows3(i):
    return (0, i, 0)


def _tc1(degp, x, W, Wl, bl):
    def body(degp_r, x_r, w_r, wl_r, bl_r, y_r, skip_r, dinv_r):
        deg = degp_r[0] + degp_r[1] + 1.0
        dinv = lax.rsqrt(deg)
        xb = x_r[...]
        y_r[...] = dinv * jnp.dot(xb, w_r[...], preferred_element_type=jnp.float32)
        skip_r[...] = jnp.dot(xb, wl_r[...], preferred_element_type=jnp.float32) + bl_r[...]
        dinv_r[...] = dinv

    return pl.pallas_call(
        body,
        grid=(GRID,),
        in_specs=[
            pl.BlockSpec((NC, BN, 1), _rows3),
            pl.BlockSpec((BN, D_IN), _rows),
            pl.BlockSpec((D_IN, D_MID), _fixed),
            pl.BlockSpec((D_IN, D_MID), _fixed),
            pl.BlockSpec((1, D_MID), _fixed),
        ],
        out_specs=[
            pl.BlockSpec((BN, D_MID), _rows),
            pl.BlockSpec((BN, D_MID), _rows),
            pl.BlockSpec((BN, 1), _rows),
        ],
        out_shape=[
            jax.ShapeDtypeStruct((N, D_MID), jnp.float32),
            jax.ShapeDtypeStruct((N, D_MID), jnp.float32),
            jax.ShapeDtypeStruct((N, 1), jnp.float32),
        ],
    )(degp, x, W, Wl, bl)


def _tc_mid(u, y, dinv, skip, b, W, Wl, bl):
    def body(u_r, y_r, dinv_r, skip_r, b_r, w_r, wl_r, bl_r, y2_r, skip2_r):
        dinv = dinv_r[...]
        h = _elu(dinv * (u_r[0] + u_r[1] + y_r[...]) + b_r[...]) + skip_r[...]
        y2_r[...] = dinv * jnp.dot(h, w_r[...], preferred_element_type=jnp.float32)
        skip2_r[...] = jnp.dot(h, wl_r[...], preferred_element_type=jnp.float32) + bl_r[...]

    return pl.pallas_call(
        body,
        grid=(GRID,),
        in_specs=[
            pl.BlockSpec((NC, BN, D_MID), _rows3),
            pl.BlockSpec((BN, D_MID), _rows),
            pl.BlockSpec((BN, 1), _rows),
            pl.BlockSpec((BN, D_MID), _rows),
            pl.BlockSpec((1, D_MID), _fixed),
            pl.BlockSpec((D_MID, D_MID), _fixed),
            pl.BlockSpec((D_MID, D_MID), _fixed),
            pl.BlockSpec((1, D_MID), _fixed),
        ],
        out_specs=[
            pl.BlockSpec((BN, D_MID), _rows),
            pl.BlockSpec((BN, D_MID), _rows),
        ],
        out_shape=[
            jax.ShapeDtypeStruct((N, D_MID), jnp.float32),
            jax.ShapeDtypeStruct((N, D_MID), jnp.float32),
        ],
    )(u, y, dinv, skip, b, W, Wl, bl)


def _tc_final(u, y, dinv, skip, b):
    def body(u_r, y_r, dinv_r, skip_r, b_r, o_r):
        u_ = u_r[0] + u_r[1]
        o_r[...] = dinv_r[...] * (u_ + y_r[...]) + b_r[...] + skip_r[...]

    return pl.pallas_call(
        body,
        grid=(GRID,),
        in_specs=[
            pl.BlockSpec((NC, BN, D_OUT), _rows3),
            pl.BlockSpec((BN, D_OUT), _rows),
            pl.BlockSpec((BN, 1), _rows),
            pl.BlockSpec((BN, D_OUT), _rows),
            pl.BlockSpec((1, D_OUT), _fixed),
        ],
        out_specs=pl.BlockSpec((BN, D_OUT), _rows),
        out_shape=jax.ShapeDtypeStruct((N, D_OUT), jnp.float32),
    )(u, y, dinv, skip, b)


def kernel(node_feature, adj_list, W1, b1, Wl1, bl1, W2, b2, Wl2, bl2,
           W3, b3, Wl3, bl3):
    x = node_feature
    src = adj_list[0].astype(jnp.int32)
    dst = adj_list[1].astype(jnp.int32)
    dst3 = dst.reshape(TILES, E_PER_TILE)
    sr = src.reshape(TILES, E_PER_TILE)[:, : NCHA * CHA].reshape(
        TILES, NCHA, 2, CHA // 2)
    srcp = (sr[:, :, 0, :] | (sr[:, :, 1, :] << 16)).reshape(TILES, PK)

    degp = _sc_degree(dst3)

    pad = D_MID - D_MID2
    W2p = jnp.pad(W2, ((0, 0), (0, pad)))
    Wl2p = jnp.pad(Wl2, ((0, 0), (0, pad)))
    W3p = jnp.pad(W3, ((0, pad), (0, 0)))
    Wl3p = jnp.pad(Wl3, ((0, pad), (0, 0)))
    b1r = b1.reshape(1, -1)
    b2r = jnp.pad(b2, (0, pad)).reshape(1, -1)
    b3r = b3.reshape(1, -1)
    bl1r = bl1.reshape(1, -1)
    bl2r = jnp.pad(bl2, (0, pad)).reshape(1, -1)
    bl3r = bl3.reshape(1, -1)

    y1, skip1, dinv = _tc1(degp.reshape(NC, N, 1), x, W1, Wl1, bl1r)
    u1 = _sc_agg128(y1, srcp, src, dst)
    y2, skip2 = _tc_mid(u1, y1, dinv, skip1, b1r, W2p, Wl2p, bl2r)
    u2 = _sc_agg128(y2, srcp, src, dst)
    y3, skip3 = _tc_mid(u2, y2, dinv, skip2, b2r, W3p, Wl3p, bl3r)
    u3 = _sc_agg128(y3, srcp, src, dst)
    out = _tc_final(u3, y3, dinv, skip3, b3r)
    return out


# speedup vs baseline: 1.0699x; 1.0699x over previous
import functools

import jax
import jax.numpy as jnp
from jax import lax
from jax.experimental import pallas as pl
from jax.experimental.pallas import tpu as pltpu
from jax.experimental.pallas import tpu_sc as plsc

N = 10000
E = 320000
D_IN = 128
D_MID = 128
D_MID2 = 64
D_OUT = 128

NC = 2
NS = 16
TILES = NC * NS
E_PER_TILE = E // TILES
RS = 624
TAIL = N - NS * RS
CH = 128
N_CHUNKS = E_PER_TILE // CH
ECH_TAIL = E_PER_TILE - N_CHUNKS * CH
NB = 4
CHA = 80
NCHA = E_PER_TILE // CHA
AGG_TAIL = E_PER_TILE - NCHA * CHA
PK = NCHA * CHA // 2
ZR = 16
NZ = RS // ZR

_mesh = lambda: plsc.VectorSubcoreMesh(core_axis_name="c", subcore_axis_name="s")


def _sc_degree(dst):
    @functools.partial(
        pl.kernel,
        mesh=_mesh(),
        out_type=jax.ShapeDtypeStruct((NC, N), jnp.float32),
        scratch_types=[
            pltpu.VMEM((E_PER_TILE,), jnp.int32),
            pltpu.VMEM((CH,), jnp.int32),
            pltpu.VMEM((CH,), jnp.int32),
            pltpu.VMEM((ECH_TAIL,), jnp.int32),
            pltpu.VMEM((CH,), jnp.float32),
            pltpu.VMEM((1008,), jnp.float32),
            pltpu.VMEM_SHARED((N,), jnp.float32),
            pltpu.SemaphoreType.DMA,
            pltpu.SemaphoreType.DMA,
        ],
    )
    def k(dst_hbm, out_hbm, slab, dca, dcb, dct, ones, zbuf, acc, sem_a, sem_b):
        c = lax.axis_index("c")
        s = lax.axis_index("s")
        wid = c * NS + s

        slab_cp = pltpu.async_copy(dst_hbm.at[wid], slab, sem_a)

        def zfill(j, carry):
            zbuf[pl.ds(j * 16, 16)] = jnp.zeros((16,), jnp.float32)
            return carry

        lax.fori_loop(0, 63, zfill, 0)

        def ofill(j, carry):
            ones[pl.ds(j * 16, 16)] = jnp.ones((16,), jnp.float32)
            return carry

        lax.fori_loop(0, CH // 16, ofill, 0)

        @pl.when(s < 10)
        def _():
            pltpu.sync_copy(zbuf.at[pl.ds(0, 1000)], acc.at[pl.ds(s * 1000, 1000)])

        slab_cp.wait()
        plsc.subcore_barrier()

        def cpidx(i, buf, n):
            def one(j, carry):
                buf[pl.ds(j * 16, 16)] = slab[pl.ds(i * CH + j * 16, 16)]
                return carry

            lax.fori_loop(0, n // 16, one, 0)

        def scat_start(i, buf, sem):
            cpidx(i, buf, CH)
            pltpu.async_copy(ones, acc.at[buf], sem, add=True)

        def scat_wait(buf, sem):
            pltpu.make_async_copy(dst_hbm.at[wid, pl.ds(0, CH)], buf, sem).wait()

        scat_start(0, dca, sem_a)

        def body(j, carry):
            i0 = 2 * j
            scat_start(i0 + 1, dcb, sem_b)
            scat_wait(dca, sem_a)

            @pl.when(i0 + 2 < N_CHUNKS)
            def _():
                scat_start(i0 + 2, dca, sem_a)

            scat_wait(dcb, sem_b)
            return carry

        lax.fori_loop(0, N_CHUNKS // 2, body, 0)
        dct[pl.ds(0, 16)] = slab[pl.ds(N_CHUNKS * CH, 16)]
        pltpu.sync_copy(ones.at[pl.ds(0, ECH_TAIL)], acc.at[dct], add=True)
        plsc.subcore_barrier()

        @pl.when(s == 0)
        def _():
            pltpu.sync_copy(acc, out_hbm.at[c])

    return k(dst)


def _make_sc_agg(d):
    nb = NB

    @functools.partial(
        pl.kernel,
        mesh=_mesh(),
        out_type=jax.ShapeDtypeStruct((NC, N, d), jnp.float32),
        scratch_types=(
            [pltpu.VMEM((PK,), jnp.int32)]
            + [pltpu.VMEM((CHA,), jnp.int32)] * nb
            + [pltpu.VMEM((CHA,), jnp.int32)] * nb
            + [pltpu.VMEM((max(AGG_TAIL, 8),), jnp.int32)] * 2
            + [pltpu.VMEM((CHA, d), jnp.float32)] * nb
            + [pltpu.VMEM((ZR, d), jnp.float32)]
            + [pltpu.VMEM_SHARED((N, d), jnp.float32)]
            + [pltpu.SemaphoreType.DMA] * (2 * nb + 1)
        ),
    )
    def k(y_hbm, src_hbm, src1_hbm, dst_hbm, out_hbm, *rest):
        srcv = rest[0]
        sbufs = rest[1:1 + nb]
        dbufs = rest[1 + nb:1 + 2 * nb]
        sct, dct = rest[1 + 2 * nb:3 + 2 * nb]
        rows = rest[3 + 2 * nb:3 + 3 * nb]
        zbuf = rest[3 + 3 * nb]
        acc = rest[4 + 3 * nb]
        gsems = rest[5 + 3 * nb:5 + 4 * nb]
        dsems = rest[5 + 4 * nb:5 + 5 * nb]
        zsem = rest[5 + 5 * nb]
        c = lax.axis_index("c")
        s = lax.axis_index("s")
        wid = c * NS + s

        idx_cp_s = pltpu.async_copy(src_hbm.at[wid], srcv, gsems[0])

        def zrow(r, carry):
            def zcol(j, carry2):
                zbuf[r, pl.ds(j * 16, 16)] = jnp.zeros((16,), jnp.float32)
                return carry2

            return lax.fori_loop(0, d // 16, zcol, carry)

        lax.fori_loop(0, ZR, zrow, 0)

        def zstart(t, carry):
            pltpu.async_copy(zbuf, acc.at[pl.ds(s * RS + t * ZR, ZR)], zsem)
            return carry

        lax.fori_loop(0, NZ, zstart, 0)

        @pl.when(s == 0)
        def _():
            pltpu.sync_copy(zbuf.at[pl.ds(0, TAIL)], acc.at[pl.ds(NS * RS, TAIL)])

        def zdrain(t, carry):
            pltpu.make_async_copy(y_hbm.at[pl.ds(0, ZR)], zbuf, zsem).wait()
            return carry

        lax.fori_loop(0, NZ, zdrain, 0)
        idx_cp_s.wait()
        plsc.subcore_barrier()

        def cpidx(i, buf):
            def one(g, carry):
                v = srcv[pl.ds(i * (CHA // 2) + g * 16, 16)]
                buf[pl.ds(g * 16, 16)] = v & 0xFFFF
                buf[pl.ds(CHA // 2 + g * 16, 16)] = v >> 16
                return carry

            lax.fori_loop(0, CHA // 32, one, 0)

        def stage(i, b):
            cpidx(i, sbufs[b])
            pltpu.async_copy(y_hbm.at[sbufs[b]], rows[b], gsems[b])
            pltpu.async_copy(
                dst_hbm.at[pl.ds(wid * E_PER_TILE + i * CHA, CHA)],
                dbufs[b], dsems[b])

        def finish(b):
            pltpu.make_async_copy(y_hbm.at[pl.ds(0, CHA)], rows[b], gsems[b]).wait()
            pltpu.make_async_copy(
                dst_hbm.at[pl.ds(0, CHA)], dbufs[b], dsems[b]).wait()
            pltpu.sync_copy(rows[b], acc.at[dbufs[b]], add=True)

        for b in range(nb - 1):
            stage(b, b)

        def body(j, carry):
            i0 = nb * j
            stage(i0 + nb - 1, nb - 1)
            for b in range(nb):
                finish(b)
                if b < nb - 1:
                    @pl.when(i0 + nb + b < NCHA)
                    def _(b=b, i0=i0):
                        stage(i0 + nb + b, b)
            return carry

        lax.fori_loop(0, NCHA // nb, body, 0)
        for r in range((NCHA // nb) * nb, NCHA):
            finish(r % nb)

        if AGG_TAIL:
            pltpu.sync_copy(
                src1_hbm.at[pl.ds(wid * E_PER_TILE + NCHA * CHA, AGG_TAIL)],
                sct)
            pltpu.sync_copy(
                dst_hbm.at[pl.ds(wid * E_PER_TILE + NCHA * CHA, AGG_TAIL)],
                dct)
            pltpu.async_copy(
                y_hbm.at[sct], rows[0].at[pl.ds(0, AGG_TAIL)], gsems[0]).wait()
            pltpu.sync_copy(
                rows[0].at[pl.ds(0, AGG_TAIL)], acc.at[dct], add=True)
        plsc.subcore_barrier()

        pltpu.sync_copy(
            acc.at[pl.ds(s * RS, RS)],
            out_hbm.at[c, pl.ds(s * RS, RS), :],
        )

        @pl.when(s == 0)
        def _():
            pltpu.sync_copy(
                acc.at[pl.ds(NS * RS, TAIL)],
                out_hbm.at[c, pl.ds(NS * RS, TAIL), :],
            )

    return k


_sc_agg128 = _make_sc_agg(D_MID)


BN = 2000
GRID = N // BN


def _elu(a):
    return jnp.where(a > 0.0, a, jnp.exp(jnp.minimum(a, 0.0)) - 1.0)


def _rows(i):
    return (i, 0)


def _fixed(i):
    return (0, 0)


def _rows3(i):
    return (0, i, 0)


def _tc1(degp, x, W, Wl, bl):
    def body(degp_r, x_r, w_r, wl_r, bl_r, y_r, skip_r, dinv_r):
        deg = degp_r[0] + degp_r[1] + 1.0
        dinv = lax.rsqrt(deg)
        xb = x_r[...]
        y_r[...] = dinv * jnp.dot(xb, w_r[...], preferred_element_type=jnp.float32)
        skip_r[...] = jnp.dot(xb, wl_r[...], preferred_element_type=jnp.float32) + bl_r[...]
        dinv_r[...] = dinv

    return pl.pallas_call(
        body,
        grid=(GRID,),
        in_specs=[
            pl.BlockSpec((NC, BN, 1), _rows3),
            pl.BlockSpec((BN, D_IN), _rows),
            pl.BlockSpec((D_IN, D_MID), _fixed),
            pl.BlockSpec((D_IN, D_MID), _fixed),
            pl.BlockSpec((1, D_MID), _fixed),
        ],
        out_specs=[
            pl.BlockSpec((BN, D_MID), _rows),
            pl.BlockSpec((BN, D_MID), _rows),
            pl.BlockSpec((BN, 1), _rows),
        ],
        out_shape=[
            jax.ShapeDtypeStruct((N, D_MID), jnp.float32),
            jax.ShapeDtypeStruct((N, D_MID), jnp.float32),
            jax.ShapeDtypeStruct((N, 1), jnp.float32),
        ],
    )(degp, x, W, Wl, bl)


def _tc_mid(u, y, dinv, skip, b, W, Wl, bl):
    def body(u_r, y_r, dinv_r, skip_r, b_r, w_r, wl_r, bl_r, y2_r, skip2_r):
        dinv = dinv_r[...]
        h = _elu(dinv * (u_r[0] + u_r[1] + y_r[...]) + b_r[...]) + skip_r[...]
        y2_r[...] = dinv * jnp.dot(h, w_r[...], preferred_element_type=jnp.float32)
        skip2_r[...] = jnp.dot(h, wl_r[...], preferred_element_type=jnp.float32) + bl_r[...]

    return pl.pallas_call(
        body,
        grid=(GRID,),
        in_specs=[
            pl.BlockSpec((NC, BN, D_MID), _rows3),
            pl.BlockSpec((BN, D_MID), _rows),
            pl.BlockSpec((BN, 1), _rows),
            pl.BlockSpec((BN, D_MID), _rows),
            pl.BlockSpec((1, D_MID), _fixed),
            pl.BlockSpec((D_MID, D_MID), _fixed),
            pl.BlockSpec((D_MID, D_MID), _fixed),
            pl.BlockSpec((1, D_MID), _fixed),
        ],
        out_specs=[
            pl.BlockSpec((BN, D_MID), _rows),
            pl.BlockSpec((BN, D_MID), _rows),
        ],
        out_shape=[
            jax.ShapeDtypeStruct((N, D_MID), jnp.float32),
            jax.ShapeDtypeStruct((N, D_MID), jnp.float32),
        ],
    )(u, y, dinv, skip, b, W, Wl, bl)


def _tc_final(u, y, dinv, skip, b):
    def body(u_r, y_r, dinv_r, skip_r, b_r, o_r):
        u_ = u_r[0] + u_r[1]
        o_r[...] = dinv_r[...] * (u_ + y_r[...]) + b_r[...] + skip_r[...]

    return pl.pallas_call(
        body,
        grid=(GRID,),
        in_specs=[
            pl.BlockSpec((NC, BN, D_OUT), _rows3),
            pl.BlockSpec((BN, D_OUT), _rows),
            pl.BlockSpec((BN, 1), _rows),
            pl.BlockSpec((BN, D_OUT), _rows),
            pl.BlockSpec((1, D_OUT), _fixed),
        ],
        out_specs=pl.BlockSpec((BN, D_OUT), _rows),
        out_shape=jax.ShapeDtypeStruct((N, D_OUT), jnp.float32),
    )(u, y, dinv, skip, b)


def kernel(node_feature, adj_list, W1, b1, Wl1, bl1, W2, b2, Wl2, bl2,
           W3, b3, Wl3, bl3):
    x = node_feature
    src = adj_list[0].astype(jnp.int32)
    dst = adj_list[1].astype(jnp.int32)
    dst3 = dst.reshape(TILES, E_PER_TILE)
    sr = src.reshape(TILES, E_PER_TILE)[:, : NCHA * CHA].reshape(
        TILES, NCHA, 2, CHA // 2)
    srcp = (sr[:, :, 0, :] | (sr[:, :, 1, :] << 16)).reshape(TILES, PK)

    degp = _sc_degree(dst3)

    pad = D_MID - D_MID2
    W2p = jnp.pad(W2, ((0, 0), (0, pad)))
    Wl2p = jnp.pad(Wl2, ((0, 0), (0, pad)))
    W3p = jnp.pad(W3, ((0, pad), (0, 0)))
    Wl3p = jnp.pad(Wl3, ((0, pad), (0, 0)))
    b1r = b1.reshape(1, -1)
    b2r = jnp.pad(b2, (0, pad)).reshape(1, -1)
    b3r = b3.reshape(1, -1)
    bl1r = bl1.reshape(1, -1)
    bl2r = jnp.pad(bl2, (0, pad)).reshape(1, -1)
    bl3r = bl3.reshape(1, -1)

    y1, skip1, dinv = _tc1(degp.reshape(NC, N, 1), x, W1, Wl1, bl1r)
    u1 = _sc_agg128(y1, srcp, src, dst)
    y2, skip2 = _tc_mid(u1, y1, dinv, skip1, b1r, W2p, Wl2p, bl2r)
    u2 = _sc_agg128(y2, srcp, src, dst)
    y3, skip3 = _tc_mid(u2, y2, dinv, skip2, b2r, W3p, Wl3p, bl3r)
    u3 = _sc_agg128(y3, srcp, src, dst)
    out = _tc_final(u3, y3, dinv, skip3, b3r)
    return out


# speedup vs baseline: 1.0887x; 1.0176x over previous
import functools

import jax
import jax.numpy as jnp
from jax import lax
from jax.experimental import pallas as pl
from jax.experimental.pallas import tpu as pltpu
from jax.experimental.pallas import tpu_sc as plsc

N = 10000
E = 320000
D_IN = 128
D_MID = 128
D_MID2 = 64
D_OUT = 128

NC = 2
NS = 16
TILES = NC * NS
E_PER_TILE = E // TILES
RS = 624
TAIL = N - NS * RS
CH = 128
N_CHUNKS = E_PER_TILE // CH
ECH_TAIL = E_PER_TILE - N_CHUNKS * CH
NB = 4
CHA = 64
NCHA = E_PER_TILE // CHA
AGG_TAIL = E_PER_TILE - NCHA * CHA
ZR = 16
NZ = RS // ZR

_mesh = lambda: plsc.VectorSubcoreMesh(core_axis_name="c", subcore_axis_name="s")


def _sc_degree(dst):
    @functools.partial(
        pl.kernel,
        mesh=_mesh(),
        out_type=jax.ShapeDtypeStruct((NC, N), jnp.float32),
        scratch_types=[
            pltpu.VMEM((E_PER_TILE,), jnp.int32),
            pltpu.VMEM((CH,), jnp.int32),
            pltpu.VMEM((CH,), jnp.int32),
            pltpu.VMEM((ECH_TAIL,), jnp.int32),
            pltpu.VMEM((CH,), jnp.float32),
            pltpu.VMEM((1008,), jnp.float32),
            pltpu.VMEM_SHARED((N,), jnp.float32),
            pltpu.SemaphoreType.DMA,
            pltpu.SemaphoreType.DMA,
        ],
    )
    def k(dst_hbm, out_hbm, slab, dca, dcb, dct, ones, zbuf, acc, sem_a, sem_b):
        c = lax.axis_index("c")
        s = lax.axis_index("s")
        wid = c * NS + s

        slab_cp = pltpu.async_copy(dst_hbm.at[wid], slab, sem_a)

        def zfill(j, carry):
            zbuf[pl.ds(j * 16, 16)] = jnp.zeros((16,), jnp.float32)
            return carry

        lax.fori_loop(0, 63, zfill, 0)

        def ofill(j, carry):
            ones[pl.ds(j * 16, 16)] = jnp.ones((16,), jnp.float32)
            return carry

        lax.fori_loop(0, CH // 16, ofill, 0)

        @pl.when(s < 10)
        def _():
            pltpu.sync_copy(zbuf.at[pl.ds(0, 1000)], acc.at[pl.ds(s * 1000, 1000)])

        slab_cp.wait()
        plsc.subcore_barrier()

        def cpidx(i, buf, n):
            def one(j, carry):
                buf[pl.ds(j * 16, 16)] = slab[pl.ds(i * CH + j * 16, 16)]
                return carry

            lax.fori_loop(0, n // 16, one, 0)

        def scat_start(i, buf, sem):
            cpidx(i, buf, CH)
            pltpu.async_copy(ones, acc.at[buf], sem, add=True)

        def scat_wait(buf, sem):
            pltpu.make_async_copy(dst_hbm.at[wid, pl.ds(0, CH)], buf, sem).wait()

        scat_start(0, dca, sem_a)

        def body(j, carry):
            i0 = 2 * j
            scat_start(i0 + 1, dcb, sem_b)
            scat_wait(dca, sem_a)

            @pl.when(i0 + 2 < N_CHUNKS)
            def _():
                scat_start(i0 + 2, dca, sem_a)

            scat_wait(dcb, sem_b)
            return carry

        lax.fori_loop(0, N_CHUNKS // 2, body, 0)
        dct[pl.ds(0, 16)] = slab[pl.ds(N_CHUNKS * CH, 16)]
        pltpu.sync_copy(ones.at[pl.ds(0, ECH_TAIL)], acc.at[dct], add=True)
        plsc.subcore_barrier()

        @pl.when(s == 0)
        def _():
            pltpu.sync_copy(acc, out_hbm.at[c])

    return k(dst)


def _make_sc_agg(d):
    nb = NB

    @functools.partial(
        pl.kernel,
        mesh=_mesh(),
        out_type=jax.ShapeDtypeStruct((NC, N, d), jnp.float32),
        scratch_types=(
            [pltpu.VMEM((E_PER_TILE,), jnp.int32)]
            + [pltpu.VMEM((CHA,), jnp.int32)] * nb
            + [pltpu.VMEM((CHA,), jnp.int32)] * nb
            + [pltpu.VMEM((AGG_TAIL,), jnp.int32)] * 2
            + [pltpu.VMEM((CHA, d), jnp.float32)] * nb
            + [pltpu.VMEM((ZR, d), jnp.float32)]
            + [pltpu.VMEM_SHARED((N, d), jnp.float32)]
            + [pltpu.SemaphoreType.DMA] * (2 * nb + 1)
        ),
    )
    def k(y_hbm, src_hbm, dst_hbm, out_hbm, *rest):
        srcv = rest[0]
        sbufs = rest[1:1 + nb]
        dbufs = rest[1 + nb:1 + 2 * nb]
        sct, dct = rest[1 + 2 * nb:3 + 2 * nb]
        rows = rest[3 + 2 * nb:3 + 3 * nb]
        zbuf = rest[3 + 3 * nb]
        acc = rest[4 + 3 * nb]
        gsems = rest[5 + 3 * nb:5 + 4 * nb]
        dsems = rest[5 + 4 * nb:5 + 5 * nb]
        zsem = rest[5 + 5 * nb]
        c = lax.axis_index("c")
        s = lax.axis_index("s")
        wid = c * NS + s

        idx_cp_s = pltpu.async_copy(src_hbm.at[wid], srcv, gsems[0])

        def zrow(r, carry):
            def zcol(j, carry2):
                zbuf[r, pl.ds(j * 16, 16)] = jnp.zeros((16,), jnp.float32)
                return carry2

            return lax.fori_loop(0, d // 16, zcol, carry)

        lax.fori_loop(0, ZR, zrow, 0)

        def zstart(t, carry):
            pltpu.async_copy(zbuf, acc.at[pl.ds(s * RS + t * ZR, ZR)], zsem)
            return carry

        lax.fori_loop(0, NZ, zstart, 0)

        @pl.when(s == 0)
        def _():
            pltpu.sync_copy(zbuf.at[pl.ds(0, TAIL)], acc.at[pl.ds(NS * RS, TAIL)])

        def zdrain(t, carry):
            pltpu.make_async_copy(y_hbm.at[pl.ds(0, ZR)], zbuf, zsem).wait()
            return carry

        lax.fori_loop(0, NZ, zdrain, 0)
        idx_cp_s.wait()
        plsc.subcore_barrier()

        def cpidx(i, buf):
            def one(j, carry):
                buf[pl.ds(j * 16, 16)] = srcv[pl.ds(i * CHA + j * 16, 16)]
                return carry

            lax.fori_loop(0, CHA // 16, one, 0)

        def stage(i, b):
            cpidx(i, sbufs[b])
            pltpu.async_copy(y_hbm.at[sbufs[b]], rows[b], gsems[b])
            pltpu.async_copy(
                dst_hbm.at[pl.ds(wid * E_PER_TILE + i * CHA, CHA)],
                dbufs[b], dsems[b])

        def finish(b):
            pltpu.make_async_copy(y_hbm.at[pl.ds(0, CHA)], rows[b], gsems[b]).wait()
            pltpu.make_async_copy(
                dst_hbm.at[pl.ds(0, CHA)], dbufs[b], dsems[b]).wait()
            pltpu.sync_copy(rows[b], acc.at[dbufs[b]], add=True)

        for b in range(nb - 1):
            stage(b, b)

        def body(j, carry):
            i0 = nb * j
            stage(i0 + nb - 1, nb - 1)
            for b in range(nb):
                finish(b)
                if b < nb - 1:
                    @pl.when(i0 + nb + b < NCHA)
                    def _(b=b, i0=i0):
                        stage(i0 + nb + b, b)
            return carry

        lax.fori_loop(0, NCHA // nb, body, 0)

        def tcp(j, carry):
            sct[pl.ds(j * 16, 16)] = srcv[pl.ds(NCHA * CHA + j * 16, 16)]
            return carry

        lax.fori_loop(0, AGG_TAIL // 16, tcp, 0)
        pltpu.sync_copy(
            dst_hbm.at[pl.ds(wid * E_PER_TILE + NCHA * CHA, AGG_TAIL)], dct)
        pltpu.async_copy(
            y_hbm.at[sct], rows[0].at[pl.ds(0, AGG_TAIL)], gsems[0]).wait()
        pltpu.sync_copy(rows[0].at[pl.ds(0, AGG_TAIL)], acc.at[dct], add=True)
        plsc.subcore_barrier()

        pltpu.sync_copy(
            acc.at[pl.ds(s * RS, RS)],
            out_hbm.at[c, pl.ds(s * RS, RS), :],
        )

        @pl.when(s == 0)
        def _():
            pltpu.sync_copy(
                acc.at[pl.ds(NS * RS, TAIL)],
                out_hbm.at[c, pl.ds(NS * RS, TAIL), :],
            )

    return k


_sc_agg128 = _make_sc_agg(D_MID)


BN = 2000
GRID = N // BN


def _elu(a):
    return jnp.where(a > 0.0, a, jnp.exp(jnp.minimum(a, 0.0)) - 1.0)


def _rows(i):
    return (i, 0)


def _fixed(i):
    return (0, 0)


def _rows3(i):
    return (0, i, 0)


def _tc1(degp, x, W, Wl, bl):
    def body(degp_r, x_r, w_r, wl_r, bl_r, y_r, skip_r, dinv_r):
        deg = degp_r[0] + degp_r[1] + 1.0
        dinv = lax.rsqrt(deg)
        xb = x_r[...]
        y_r[...] = dinv * jnp.dot(xb, w_r[...], preferred_element_type=jnp.float32)
        skip_r[...] = jnp.dot(xb, wl_r[...], preferred_element_type=jnp.float32) + bl_r[...]
        dinv_r[...] = dinv

    return pl.pallas_call(
        body,
        grid=(GRID,),
        in_specs=[
            pl.BlockSpec((NC, BN, 1), _rows3),
            pl.BlockSpec((BN, D_IN), _rows),
            pl.BlockSpec((D_IN, D_MID), _fixed),
            pl.BlockSpec((D_IN, D_MID), _fixed),
            pl.BlockSpec((1, D_MID), _fixed),
        ],
        out_specs=[
            pl.BlockSpec((BN, D_MID), _rows),
            pl.BlockSpec((BN, D_MID), _rows),
            pl.BlockSpec((BN, 1), _rows),
        ],
        out_shape=[
            jax.ShapeDtypeStruct((N, D_MID), jnp.float32),
            jax.ShapeDtypeStruct((N, D_MID), jnp.float32),
            jax.ShapeDtypeStruct((N, 1), jnp.float32),
        ],
    )(degp, x, W, Wl, bl)


def _tc_mid(u, y, dinv, skip, b, W, Wl, bl):
    def body(u_r, y_r, dinv_r, skip_r, b_r, w_r, wl_r, bl_r, y2_r, skip2_r):
        dinv = dinv_r[...]
        h = _elu(dinv * (u_r[0] + u_r[1] + y_r[...]) + b_r[...]) + skip_r[...]
        y2_r[...] = dinv * jnp.dot(h, w_r[...], preferred_element_type=jnp.float32)
        skip2_r[...] = jnp.dot(h, wl_r[...], preferred_element_type=jnp.float32) + bl_r[...]

    return pl.pallas_call(
        body,
        grid=(GRID,),
        in_specs=[
            pl.BlockSpec((NC, BN, D_MID), _rows3),
            pl.BlockSpec((BN, D_MID), _rows),
            pl.BlockSpec((BN, 1), _rows),
            pl.BlockSpec((BN, D_MID), _rows),
            pl.BlockSpec((1, D_MID), _fixed),
            pl.BlockSpec((D_MID, D_MID), _fixed),
            pl.BlockSpec((D_MID, D_MID), _fixed),
            pl.BlockSpec((1, D_MID), _fixed),
        ],
        out_specs=[
            pl.BlockSpec((BN, D_MID), _rows),
            pl.BlockSpec((BN, D_MID), _rows),
        ],
        out_shape=[
            jax.ShapeDtypeStruct((N, D_MID), jnp.float32),
            jax.ShapeDtypeStruct((N, D_MID), jnp.float32),
        ],
    )(u, y, dinv, skip, b, W, Wl, bl)


def _tc_final(u, y, dinv, skip, b):
    def body(u_r, y_r, dinv_r, skip_r, b_r, o_r):
        u_ = u_r[0] + u_r[1]
        o_r[...] = dinv_r[...] * (u_ + y_r[...]) + b_r[...] + skip_r[...]

    return pl.pallas_call(
        body,
        grid=(GRID,),
        in_specs=[
            pl.BlockSpec((NC, BN, D_OUT), _rows3),
            pl.BlockSpec((BN, D_OUT), _rows),
            pl.BlockSpec((BN, 1), _rows),
            pl.BlockSpec((BN, D_OUT), _rows),
            pl.BlockSpec((1, D_OUT), _fixed),
        ],
        out_specs=pl.BlockSpec((BN, D_OUT), _rows),
        out_shape=jax.ShapeDtypeStruct((N, D_OUT), jnp.float32),
    )(u, y, dinv, skip, b)


def kernel(node_feature, adj_list, W1, b1, Wl1, bl1, W2, b2, Wl2, bl2,
           W3, b3, Wl3, bl3):
    x = node_feature
    src = adj_list[0].astype(jnp.int32)
    dst = adj_list[1].astype(jnp.int32)
    src3 = src.reshape(TILES, E_PER_TILE)
    dst3 = dst.reshape(TILES, E_PER_TILE)

    degp = _sc_degree(dst3)

    pad = D_MID - D_MID2
    W2p = jnp.pad(W2, ((0, 0), (0, pad)))
    Wl2p = jnp.pad(Wl2, ((0, 0), (0, pad)))
    W3p = jnp.pad(W3, ((0, pad), (0, 0)))
    Wl3p = jnp.pad(Wl3, ((0, pad), (0, 0)))
    b1r = b1.reshape(1, -1)
    b2r = jnp.pad(b2, (0, pad)).reshape(1, -1)
    b3r = b3.reshape(1, -1)
    bl1r = bl1.reshape(1, -1)
    bl2r = jnp.pad(bl2, (0, pad)).reshape(1, -1)
    bl3r = bl3.reshape(1, -1)

    y1, skip1, dinv = _tc1(degp.reshape(NC, N, 1), x, W1, Wl1, bl1r)
    u1 = _sc_agg128(y1, src3, dst)
    y2, skip2 = _tc_mid(u1, y1, dinv, skip1, b1r, W2p, Wl2p, bl2r)
    u2 = _sc_agg128(y2, src3, dst)
    y3, skip3 = _tc_mid(u2, y2, dinv, skip2, b2r, W3p, Wl3p, bl3r)
    u3 = _sc_agg128(y3, src3, dst)
    out = _tc_final(u3, y3, dinv, skip3, b3r)
    return out
